# Initial kernel scaffold; baseline (speedup 1.0000x reference)
import jax
import jax.numpy as jnp
from jax.experimental import pallas as pl


def kernel(x, table):
    raise NotImplementedError("write your pallas kernel here")


# speedup vs baseline: 3.1683x; 3.1683x over previous
import functools

import jax
import jax.numpy as jnp
from jax import lax
from jax.experimental import pallas as pl
from jax.experimental.pallas import tpu as pltpu
from jax.experimental.pallas import tpu_sc as plsc

_VOCAB = 100000
_VOCAB_PAD = 102400
_EMBED = 32
_B = 4
_SPATIAL = 64 * 64 * 64
_CHUNK = 4096
_CHUNKS_PER_BATCH = _SPATIAL // _CHUNK
_NCHUNKS = _B * _CHUNKS_PER_BATCH


def _transpose_table(table):
    rows = 2048
    grid = _VOCAB_PAD // rows

    def body(in_ref, out_ref):
        out_ref[...] = in_ref[...].T

    return pl.pallas_call(
        body,
        grid=(grid,),
        in_specs=[pl.BlockSpec((rows, _EMBED), lambda g: (g, 0))],
        out_specs=pl.BlockSpec((_EMBED, rows), lambda g: (0, g)),
        out_shape=jax.ShapeDtypeStruct((_EMBED, _VOCAB_PAD), jnp.float32),
    )(table)


def _sc_lookup(table_t, idx):
    mesh = plsc.VectorSubcoreMesh(core_axis_name="c", subcore_axis_name="s")

    @functools.partial(
        pl.kernel,
        mesh=mesh,
        compiler_params=pltpu.CompilerParams(
            use_tc_tiling_on_sc=False, needs_layout_passes=False
        ),
        out_type=jax.ShapeDtypeStruct((_B * _EMBED, _SPATIAL), jnp.float32),
        scratch_types=[
            pltpu.VMEM((_VOCAB_PAD,), jnp.float32),
            pltpu.VMEM((_CHUNK,), jnp.int32),
            pltpu.VMEM((_CHUNK,), jnp.float32),
        ],
    )
    def k(table_t_hbm, idx_hbm, out_hbm, row_v, idx_v, val_v):
        e = lax.axis_index("s") * 2 + lax.axis_index("c")
        pltpu.sync_copy(table_t_hbm.at[e], row_v)

        def chunk_body(g, carry):
            pltpu.sync_copy(idx_hbm.at[pl.ds(g * _CHUNK, _CHUNK)], idx_v)

            def gather_body(i, carry2):
                ids = idx_v[pl.ds(i * 16, 16)]
                val_v[pl.ds(i * 16, 16)] = plsc.load_gather(row_v, [ids])
                return carry2

            lax.fori_loop(0, _CHUNK // 16, gather_body, 0, unroll=4)
            b = g // _CHUNKS_PER_BATCH
            s0 = (g % _CHUNKS_PER_BATCH) * _CHUNK
            pltpu.sync_copy(val_v, out_hbm.at[b * _EMBED + e, pl.ds(s0, _CHUNK)])
            return carry

        lax.fori_loop(0, _NCHUNKS, chunk_body, 0)

    return k(table_t, idx)


def kernel(x, table):
    idx = x[:, 0].reshape(_B * _SPATIAL).astype(jnp.int32)
    table_pad = jnp.pad(table, ((0, _VOCAB_PAD - _VOCAB), (0, 0)))
    table_t = _transpose_table(table_pad)
    out = _sc_lookup(table_t, idx)
    return out.reshape(_B, _EMBED, 64, 64, 64)


# speedup vs baseline: 4.2563x; 1.3434x over previous
import functools

import jax
import jax.numpy as jnp
from jax import lax
from jax.experimental import pallas as pl
from jax.experimental.pallas import tpu as pltpu
from jax.experimental.pallas import tpu_sc as plsc

_VOCAB = 100000
_VOCAB_PAD = 102400
_EMBED = 32
_B = 4
_SPATIAL = 64 * 64 * 64
_CHUNK = 4096
_CHUNKS_PER_BATCH = _SPATIAL // _CHUNK
_NCHUNKS = _B * _CHUNKS_PER_BATCH


def _transpose_table(table):
    rows = 4096
    grid = _VOCAB_PAD // rows

    def body(in_ref, out_ref):
        out_ref[...] = in_ref[...].T

    return pl.pallas_call(
        body,
        grid=(grid,),
        in_specs=[pl.BlockSpec((rows, _EMBED), lambda g: (g, 0))],
        out_specs=pl.BlockSpec((_EMBED, rows), lambda g: (0, g)),
        out_shape=jax.ShapeDtypeStruct((_EMBED, _VOCAB_PAD), jnp.float32),
    )(table)


def _sc_lookup(table_t, idx):
    mesh = plsc.VectorSubcoreMesh(core_axis_name="c", subcore_axis_name="s")

    @functools.partial(
        pl.kernel,
        mesh=mesh,
        compiler_params=pltpu.CompilerParams(
            use_tc_tiling_on_sc=False, needs_layout_passes=False
        ),
        out_type=jax.ShapeDtypeStruct((_B * _EMBED, _SPATIAL), jnp.float32),
        scratch_types=[
            pltpu.VMEM((_VOCAB,), jnp.float32),
            pltpu.VMEM((2, _CHUNK), jnp.int32),
            pltpu.VMEM((2, _CHUNK), jnp.float32),
            pltpu.SemaphoreType.DMA,
            (pltpu.SemaphoreType.DMA,) * 2,
            (pltpu.SemaphoreType.DMA,) * 2,
        ],
    )
    def k(table_t_hbm, idx_hbm, out_hbm, row_v, idx_v, val_v,
          row_sem, in_sems, out_sems):
        e = lax.axis_index("s") * 2 + lax.axis_index("c")
        row_copy = pltpu.make_async_copy(
            table_t_hbm.at[e, pl.ds(0, _VOCAB)], row_v, row_sem
        )
        row_copy.start()

        def start_in(g, buf, sem):
            pltpu.make_async_copy(
                idx_hbm.at[pl.ds(g * _CHUNK, _CHUNK)], buf, sem
            ).start()

        def wait_in(g, buf, sem):
            pltpu.make_async_copy(
                idx_hbm.at[pl.ds(g * _CHUNK, _CHUNK)], buf, sem
            ).wait()

        def out_slot(g):
            b = g // _CHUNKS_PER_BATCH
            s0 = (g % _CHUNKS_PER_BATCH) * _CHUNK
            return out_hbm.at[b * _EMBED + e, pl.ds(s0, _CHUNK)]

        def start_out(g, buf, sem):
            pltpu.make_async_copy(buf, out_slot(g), sem).start()

        def wait_out(g, buf, sem):
            pltpu.make_async_copy(buf, out_slot(g), sem).wait()

        def gather(ibuf, vbuf):
            def gather_body(i, carry):
                ids = ibuf[pl.ds(i * 16, 16)]
                vbuf[pl.ds(i * 16, 16)] = plsc.load_gather(row_v, [ids])
                return carry

            lax.fori_loop(0, _CHUNK // 16, gather_body, 0, unroll=8)

        start_in(0, idx_v.at[0], in_sems[0])
        row_copy.wait()
        start_in(1, idx_v.at[1], in_sems[1])

        def chunk_pair(h, carry):
            g0 = 2 * h

            @pl.when(h > 0)
            def _():
                wait_out(g0 - 2, val_v.at[0], out_sems[0])
            wait_in(g0, idx_v.at[0], in_sems[0])
            gather(idx_v.at[0], val_v.at[0])
            start_out(g0, val_v.at[0], out_sems[0])

            @pl.when(h + 1 < _NCHUNKS // 2)
            def _():
                start_in(g0 + 2, idx_v.at[0], in_sems[0])

                @pl.when(h > 0)
                def _():
                    wait_out(g0 - 1, val_v.at[1], out_sems[1])
                wait_in(g0 + 1, idx_v.at[1], in_sems[1])
                gather(idx_v.at[1], val_v.at[1])
                start_out(g0 + 1, val_v.at[1], out_sems[1])
                start_in(g0 + 3, idx_v.at[1], in_sems[1])

            @pl.when(h + 1 == _NCHUNKS // 2)
            def _():
                wait_out(g0 - 1, val_v.at[1], out_sems[1])
                wait_in(g0 + 1, idx_v.at[1], in_sems[1])
                gather(idx_v.at[1], val_v.at[1])
                start_out(g0 + 1, val_v.at[1], out_sems[1])
                wait_out(g0, val_v.at[0], out_sems[0])
                wait_out(g0 + 1, val_v.at[1], out_sems[1])
            return carry

        lax.fori_loop(0, _NCHUNKS // 2, chunk_pair, 0)

    return k(table_t, idx)


def kernel(x, table):
    idx = x[:, 0].reshape(_B * _SPATIAL).astype(jnp.int32)
    table_t = _transpose_table(table)
    out = _sc_lookup(table_t, idx)
    return out.reshape(_B, _EMBED, 64, 64, 64)


# speedup vs baseline: 5.5281x; 1.2988x over previous
import functools

import jax
import jax.numpy as jnp
from jax import lax
from jax.experimental import pallas as pl
from jax.experimental.pallas import tpu as pltpu
from jax.experimental.pallas import tpu_sc as plsc

_VOCAB = 100000
_VOCAB_PAD = 102400
_EMBED = 32
_B = 4
_D = 64
_HW = 64 * 64
_SPATIAL = _D * _HW
_NCHUNKS = _B * _D


def _transpose_table(table):
    rows = 4096
    grid = _VOCAB_PAD // rows

    def body(in_ref, out_ref):
        out_ref[...] = in_ref[...].T.reshape(_EMBED, rows // 128, 128)

    return pl.pallas_call(
        body,
        grid=(grid,),
        in_specs=[pl.BlockSpec((rows, _EMBED), lambda g: (g, 0))],
        out_specs=pl.BlockSpec(
            (_EMBED, rows // 128, 128), lambda g: (0, g, 0)
        ),
        out_shape=jax.ShapeDtypeStruct(
            (_EMBED, _VOCAB_PAD // 128, 128), jnp.float32
        ),
    )(table)


def _sc_lookup(table_t, idx):
    mesh = plsc.VectorSubcoreMesh(core_axis_name="c", subcore_axis_name="s")

    @functools.partial(
        pl.kernel,
        mesh=mesh,
        compiler_params=pltpu.CompilerParams(
            use_tc_tiling_on_sc=True, needs_layout_passes=False
        ),
        out_type=jax.ShapeDtypeStruct((_B, _EMBED, _D, 64, 64), jnp.float32),
        scratch_types=[
            pltpu.VMEM((_VOCAB_PAD // 128, 128), jnp.float32),
            pltpu.VMEM((_HW,), jnp.int32),
            pltpu.VMEM((_HW,), jnp.int32),
            pltpu.VMEM((64, 64), jnp.float32),
            pltpu.VMEM((64, 64), jnp.float32),
            pltpu.SemaphoreType.DMA,
            (pltpu.SemaphoreType.DMA,) * 2,
            (pltpu.SemaphoreType.DMA,) * 2,
        ],
    )
    def k(table_t_hbm, idx_hbm, out_hbm, row_v, idx_v0, idx_v1,
          val_v0, val_v1, row_sem, in_sems, out_sems):
        e = lax.axis_index("s") * 2 + lax.axis_index("c")
        row_copy = pltpu.make_async_copy(table_t_hbm.at[e], row_v, row_sem)
        row_copy.start()
        idx_bufs = (idx_v0, idx_v1)
        val_bufs = (val_v0, val_v1)

        def start_in(g, buf, sem):
            pltpu.make_async_copy(
                idx_hbm.at[pl.ds(g * _HW, _HW)], buf, sem
            ).start()

        def wait_in(g, buf, sem):
            pltpu.make_async_copy(
                idx_hbm.at[pl.ds(g * _HW, _HW)], buf, sem
            ).wait()

        def out_slot(g):
            b = g // _D
            d = g % _D
            return out_hbm.at[b, e, d]

        def start_out(g, buf, sem):
            pltpu.make_async_copy(buf, out_slot(g), sem).start()

        def wait_out(g, buf, sem):
            pltpu.make_async_copy(buf, out_slot(g), sem).wait()

        def gather(ibuf, vbuf):
            def gather_body(i, carry):
                ids = ibuf[pl.ds(i * 16, 16)]
                h = i // 4
                w0 = (i % 4) * 16
                vbuf[h, pl.ds(w0, 16)] = plsc.load_gather(
                    row_v, [ids >> 7, ids & 127]
                )
                return carry

            lax.fori_loop(0, _HW // 16, gather_body, 0, unroll=8)

        start_in(0, idx_bufs[0], in_sems[0])
        row_copy.wait()
        start_in(1, idx_bufs[1], in_sems[1])

        def chunk_pair(h, carry):
            g0 = 2 * h

            @pl.when(h > 0)
            def _():
                wait_out(g0 - 2, val_bufs[0], out_sems[0])
            wait_in(g0, idx_bufs[0], in_sems[0])
            gather(idx_bufs[0], val_bufs[0])
            start_out(g0, val_bufs[0], out_sems[0])

            @pl.when(h + 1 < _NCHUNKS // 2)
            def _():
                start_in(g0 + 2, idx_bufs[0], in_sems[0])

                @pl.when(h > 0)
                def _():
                    wait_out(g0 - 1, val_bufs[1], out_sems[1])
                wait_in(g0 + 1, idx_bufs[1], in_sems[1])
                gather(idx_bufs[1], val_bufs[1])
                start_out(g0 + 1, val_bufs[1], out_sems[1])
                start_in(g0 + 3, idx_bufs[1], in_sems[1])

            @pl.when(h + 1 == _NCHUNKS // 2)
            def _():
                wait_out(g0 - 1, val_bufs[1], out_sems[1])
                wait_in(g0 + 1, idx_bufs[1], in_sems[1])
                gather(idx_bufs[1], val_bufs[1])
                start_out(g0 + 1, val_bufs[1], out_sems[1])
                wait_out(g0, val_bufs[0], out_sems[0])
                wait_out(g0 + 1, val_bufs[1], out_sems[1])
            return carry

        lax.fori_loop(0, _NCHUNKS // 2, chunk_pair, 0)

    return k(table_t, idx)


def kernel(x, table):
    idx = x[:, 0].reshape(_B * _SPATIAL).astype(jnp.int32)
    table_t = _transpose_table(table)
    return _sc_lookup(table_t, idx)


# speedup vs baseline: 11.8647x; 2.1463x over previous
import functools

import jax
import jax.numpy as jnp
from jax import lax
from jax.experimental import pallas as pl
from jax.experimental.pallas import tpu as pltpu
from jax.experimental.pallas import tpu_sc as plsc

_VOCAB = 100000
_VOCAB_PAD = 102400
_EMBED = 32
_B = 4
_D = 64
_HW = 64 * 64
_SPATIAL = _D * _HW
_NCHUNKS = _B * _D


def _transpose_table(table):
    rows = 4096
    grid = _VOCAB_PAD // rows

    def body(in_ref, out_ref):
        out_ref[...] = in_ref[...].T.reshape(_EMBED, rows // 128, 128)

    return pl.pallas_call(
        body,
        grid=(grid,),
        in_specs=[pl.BlockSpec((rows, _EMBED), lambda g: (g, 0))],
        out_specs=pl.BlockSpec(
            (_EMBED, rows // 128, 128), lambda g: (0, g, 0)
        ),
        out_shape=jax.ShapeDtypeStruct(
            (_EMBED, _VOCAB_PAD // 128, 128), jnp.float32
        ),
    )(table)


def _sc_lookup(table_t, idx):
    mesh = plsc.VectorSubcoreMesh(core_axis_name="c", subcore_axis_name="s")

    @functools.partial(
        pl.kernel,
        mesh=mesh,
        compiler_params=pltpu.CompilerParams(
            use_tc_tiling_on_sc=True, needs_layout_passes=False
        ),
        out_type=jax.ShapeDtypeStruct((_B, _EMBED, _D, 64, 64), jnp.float32),
        scratch_types=[
            pltpu.VMEM((_VOCAB_PAD // 128, 128), jnp.float32),
            pltpu.VMEM((_HW,), jnp.int32),
            pltpu.VMEM((_HW,), jnp.int32),
            pltpu.VMEM((64, 64), jnp.float32),
            pltpu.VMEM((64, 64), jnp.float32),
            pltpu.SemaphoreType.DMA,
            (pltpu.SemaphoreType.DMA,) * 2,
            (pltpu.SemaphoreType.DMA,) * 2,
        ],
    )
    def k(table_t_hbm, idx_hbm, out_hbm, row_v, idx_v0, idx_v1,
          val_v0, val_v1, row_sem, in_sems, out_sems):
        e = lax.axis_index("s") * 2 + lax.axis_index("c")
        row_copy = pltpu.make_async_copy(table_t_hbm.at[e], row_v, row_sem)
        row_copy.start()
        idx_bufs = (idx_v0, idx_v1)
        val_bufs = (val_v0, val_v1)

        def start_in(g, buf, sem):
            pltpu.make_async_copy(
                idx_hbm.at[pl.ds(g * _HW, _HW)], buf, sem
            ).start()

        def wait_in(g, buf, sem):
            pltpu.make_async_copy(
                idx_hbm.at[pl.ds(g * _HW, _HW)], buf, sem
            ).wait()

        def out_slot(g):
            b = g // _D
            d = g % _D
            return out_hbm.at[b, e, d]

        def start_out(g, buf, sem):
            pltpu.make_async_copy(buf, out_slot(g), sem).start()

        def wait_out(g, buf, sem):
            pltpu.make_async_copy(buf, out_slot(g), sem).wait()

        def gather(ibuf, vbuf):
            @plsc.parallel_loop(0, _HW // 16, unroll=8)
            def _(i):
                ids = ibuf[pl.ds(i * 16, 16)]
                h = i // 4
                w0 = (i % 4) * 16
                vbuf[h, pl.ds(w0, 16)] = plsc.load_gather(
                    row_v, [ids >> 7, ids & 127]
                )

        start_in(0, idx_bufs[0], in_sems[0])
        row_copy.wait()
        start_in(1, idx_bufs[1], in_sems[1])

        def chunk_pair(h, carry):
            g0 = 2 * h

            @pl.when(h > 0)
            def _():
                wait_out(g0 - 2, val_bufs[0], out_sems[0])
            wait_in(g0, idx_bufs[0], in_sems[0])
            gather(idx_bufs[0], val_bufs[0])
            start_out(g0, val_bufs[0], out_sems[0])

            @pl.when(h + 1 < _NCHUNKS // 2)
            def _():
                start_in(g0 + 2, idx_bufs[0], in_sems[0])

                @pl.when(h > 0)
                def _():
                    wait_out(g0 - 1, val_bufs[1], out_sems[1])
                wait_in(g0 + 1, idx_bufs[1], in_sems[1])
                gather(idx_bufs[1], val_bufs[1])
                start_out(g0 + 1, val_bufs[1], out_sems[1])
                start_in(g0 + 3, idx_bufs[1], in_sems[1])

            @pl.when(h + 1 == _NCHUNKS // 2)
            def _():
                wait_out(g0 - 1, val_bufs[1], out_sems[1])
                wait_in(g0 + 1, idx_bufs[1], in_sems[1])
                gather(idx_bufs[1], val_bufs[1])
                start_out(g0 + 1, val_bufs[1], out_sems[1])
                wait_out(g0, val_bufs[0], out_sems[0])
                wait_out(g0 + 1, val_bufs[1], out_sems[1])
            return carry

        lax.fori_loop(0, _NCHUNKS // 2, chunk_pair, 0)

    return k(table_t, idx)


def kernel(x, table):
    idx = x[:, 0].reshape(_B * _SPATIAL).astype(jnp.int32)
    table_t = _transpose_table(table)
    return _sc_lookup(table_t, idx)


# speedup vs baseline: 11.8800x; 1.0013x over previous
import functools

import jax
import jax.numpy as jnp
from jax import lax
from jax.experimental import pallas as pl
from jax.experimental.pallas import tpu as pltpu
from jax.experimental.pallas import tpu_sc as plsc

_VOCAB = 100000
_VOCAB_PAD = 102400
_EMBED = 32
_B = 4
_D = 64
_HW = 64 * 64
_SPATIAL = _D * _HW
_NCHUNKS = _B * _D


def _transpose_table(table):
    rows = 4096
    grid = _VOCAB_PAD // rows

    def body(in_ref, out_ref):
        out_ref[...] = in_ref[...].T.reshape(_EMBED, rows // 128, 128)

    return pl.pallas_call(
        body,
        grid=(grid,),
        in_specs=[pl.BlockSpec((rows, _EMBED), lambda g: (g, 0))],
        out_specs=pl.BlockSpec(
            (_EMBED, rows // 128, 128), lambda g: (0, g, 0)
        ),
        out_shape=jax.ShapeDtypeStruct(
            (_EMBED, _VOCAB_PAD // 128, 128), jnp.float32
        ),
    )(table)


def _sc_lookup(table_t, idx):
    mesh = plsc.VectorSubcoreMesh(core_axis_name="c", subcore_axis_name="s")

    @functools.partial(
        pl.kernel,
        mesh=mesh,
        compiler_params=pltpu.CompilerParams(
            use_tc_tiling_on_sc=True, needs_layout_passes=False
        ),
        out_type=jax.ShapeDtypeStruct((_B, _EMBED, _D, 64, 64), jnp.float32),
        scratch_types=[
            pltpu.VMEM((_VOCAB_PAD // 128, 128), jnp.float32),
            pltpu.VMEM((_HW,), jnp.int32),
            pltpu.VMEM((_HW,), jnp.int32),
            pltpu.VMEM((64, 64), jnp.float32),
            pltpu.VMEM((64, 64), jnp.float32),
            pltpu.SemaphoreType.DMA,
            (pltpu.SemaphoreType.DMA,) * 2,
            (pltpu.SemaphoreType.DMA,) * 2,
        ],
    )
    def k(table_t_hbm, idx_hbm, out_hbm, row_v, idx_v0, idx_v1,
          val_v0, val_v1, row_sem, in_sems, out_sems):
        e = lax.axis_index("s") * 2 + lax.axis_index("c")
        row_copy = pltpu.make_async_copy(table_t_hbm.at[e], row_v, row_sem)
        row_copy.start()
        idx_bufs = (idx_v0, idx_v1)
        val_bufs = (val_v0, val_v1)

        def start_in(g, buf, sem):
            pltpu.make_async_copy(
                idx_hbm.at[pl.ds(g * _HW, _HW)], buf, sem
            ).start()

        def wait_in(g, buf, sem):
            pltpu.make_async_copy(
                idx_hbm.at[pl.ds(g * _HW, _HW)], buf, sem
            ).wait()

        def out_slot(g):
            b = g // _D
            d = g % _D
            return out_hbm.at[b, e, d]

        def start_out(g, buf, sem):
            pltpu.make_async_copy(buf, out_slot(g), sem).start()

        def wait_out(g, buf, sem):
            pltpu.make_async_copy(buf, out_slot(g), sem).wait()

        def gather(ibuf, vbuf):
            @plsc.parallel_loop(0, _HW // 16, unroll=16)
            def _(i):
                ids = ibuf[pl.ds(i * 16, 16)]
                h = i // 4
                w0 = (i % 4) * 16
                vbuf[h, pl.ds(w0, 16)] = plsc.load_gather(
                    row_v, [ids >> 7, ids & 127]
                )

        start_in(0, idx_bufs[0], in_sems[0])
        row_copy.wait()
        start_in(1, idx_bufs[1], in_sems[1])

        def chunk_pair(h, carry):
            g0 = 2 * h

            @pl.when(h > 0)
            def _():
                wait_out(g0 - 2, val_bufs[0], out_sems[0])
            wait_in(g0, idx_bufs[0], in_sems[0])
            gather(idx_bufs[0], val_bufs[0])
            start_out(g0, val_bufs[0], out_sems[0])

            @pl.when(h + 1 < _NCHUNKS // 2)
            def _():
                start_in(g0 + 2, idx_bufs[0], in_sems[0])

                @pl.when(h > 0)
                def _():
                    wait_out(g0 - 1, val_bufs[1], out_sems[1])
                wait_in(g0 + 1, idx_bufs[1], in_sems[1])
                gather(idx_bufs[1], val_bufs[1])
                start_out(g0 + 1, val_bufs[1], out_sems[1])
                start_in(g0 + 3, idx_bufs[1], in_sems[1])

            @pl.when(h + 1 == _NCHUNKS // 2)
            def _():
                wait_out(g0 - 1, val_bufs[1], out_sems[1])
                wait_in(g0 + 1, idx_bufs[1], in_sems[1])
                gather(idx_bufs[1], val_bufs[1])
                start_out(g0 + 1, val_bufs[1], out_sems[1])
                wait_out(g0, val_bufs[0], out_sems[0])
                wait_out(g0 + 1, val_bufs[1], out_sems[1])
            return carry

        lax.fori_loop(0, _NCHUNKS // 2, chunk_pair, 0)

    return k(table_t, idx)


def kernel(x, table):
    idx = x[:, 0].reshape(_B * _SPATIAL).astype(jnp.int32)
    table_t = _transpose_table(table)
    return _sc_lookup(table_t, idx)


# speedup vs baseline: 13.8205x; 1.1633x over previous
import functools

import jax
import jax.numpy as jnp
from jax import lax
from jax.experimental import pallas as pl
from jax.experimental.pallas import tpu as pltpu
from jax.experimental.pallas import tpu_sc as plsc

_VOCAB = 100000
_VOCAB_PAD = 102400
_EMBED = 32
_B = 4
_D = 64
_HW = 64 * 64
_SPATIAL = _D * _HW
_NCHUNKS = _B * _D


def _retile_table(table_t2d):
    cols = 4096
    grid = _VOCAB_PAD // cols

    def body(in_ref, out_ref):
        out_ref[...] = in_ref[...].reshape(_EMBED, cols // 128, 128)

    return pl.pallas_call(
        body,
        grid=(grid,),
        in_specs=[pl.BlockSpec((_EMBED, cols), lambda g: (0, g))],
        out_specs=pl.BlockSpec(
            (_EMBED, cols // 128, 128), lambda g: (0, g, 0)
        ),
        out_shape=jax.ShapeDtypeStruct(
            (_EMBED, _VOCAB_PAD // 128, 128), jnp.float32
        ),
    )(table_t2d)


def _sc_lookup(table_t, idx):
    mesh = plsc.VectorSubcoreMesh(core_axis_name="c", subcore_axis_name="s")

    @functools.partial(
        pl.kernel,
        mesh=mesh,
        compiler_params=pltpu.CompilerParams(
            use_tc_tiling_on_sc=True, needs_layout_passes=False
        ),
        out_type=jax.ShapeDtypeStruct((_B, _EMBED, _D, 64, 64), jnp.float32),
        scratch_types=[
            pltpu.VMEM((_VOCAB_PAD // 128, 128), jnp.float32),
            pltpu.VMEM((_HW,), jnp.int32),
            pltpu.VMEM((_HW,), jnp.int32),
            pltpu.VMEM((64, 64), jnp.float32),
            pltpu.VMEM((64, 64), jnp.float32),
            pltpu.SemaphoreType.DMA,
            (pltpu.SemaphoreType.DMA,) * 2,
            (pltpu.SemaphoreType.DMA,) * 2,
        ],
    )
    def k(table_t_hbm, idx_hbm, out_hbm, row_v, idx_v0, idx_v1,
          val_v0, val_v1, row_sem, in_sems, out_sems):
        e = lax.axis_index("s") * 2 + lax.axis_index("c")
        row_copy = pltpu.make_async_copy(table_t_hbm.at[e], row_v, row_sem)
        row_copy.start()
        idx_bufs = (idx_v0, idx_v1)
        val_bufs = (val_v0, val_v1)

        def start_in(g, buf, sem):
            pltpu.make_async_copy(
                idx_hbm.at[pl.ds(g * _HW, _HW)], buf, sem
            ).start()

        def wait_in(g, buf, sem):
            pltpu.make_async_copy(
                idx_hbm.at[pl.ds(g * _HW, _HW)], buf, sem
            ).wait()

        def out_slot(g):
            b = g // _D
            d = g % _D
            return out_hbm.at[b, e, d]

        def start_out(g, buf, sem):
            pltpu.make_async_copy(buf, out_slot(g), sem).start()

        def wait_out(g, buf, sem):
            pltpu.make_async_copy(buf, out_slot(g), sem).wait()

        def gather(ibuf, vbuf):
            @plsc.parallel_loop(0, _HW // 16, unroll=16)
            def _(i):
                ids = ibuf[pl.ds(i * 16, 16)]
                h = i // 4
                w0 = (i % 4) * 16
                vbuf[h, pl.ds(w0, 16)] = plsc.load_gather(
                    row_v, [ids >> 7, ids & 127]
                )

        start_in(0, idx_bufs[0], in_sems[0])
        row_copy.wait()
        start_in(1, idx_bufs[1], in_sems[1])

        def chunk_pair(h, carry):
            g0 = 2 * h

            @pl.when(h > 0)
            def _():
                wait_out(g0 - 2, val_bufs[0], out_sems[0])
            wait_in(g0, idx_bufs[0], in_sems[0])
            gather(idx_bufs[0], val_bufs[0])
            start_out(g0, val_bufs[0], out_sems[0])

            @pl.when(h + 1 < _NCHUNKS // 2)
            def _():
                start_in(g0 + 2, idx_bufs[0], in_sems[0])

                @pl.when(h > 0)
                def _():
                    wait_out(g0 - 1, val_bufs[1], out_sems[1])
                wait_in(g0 + 1, idx_bufs[1], in_sems[1])
                gather(idx_bufs[1], val_bufs[1])
                start_out(g0 + 1, val_bufs[1], out_sems[1])
                start_in(g0 + 3, idx_bufs[1], in_sems[1])

            @pl.when(h + 1 == _NCHUNKS // 2)
            def _():
                wait_out(g0 - 1, val_bufs[1], out_sems[1])
                wait_in(g0 + 1, idx_bufs[1], in_sems[1])
                gather(idx_bufs[1], val_bufs[1])
                start_out(g0 + 1, val_bufs[1], out_sems[1])
                wait_out(g0, val_bufs[0], out_sems[0])
                wait_out(g0 + 1, val_bufs[1], out_sems[1])
            return carry

        lax.fori_loop(0, _NCHUNKS // 2, chunk_pair, 0)

    return k(table_t, idx)


def kernel(x, table):
    idx = x[:, 0].reshape(_B * _SPATIAL).astype(jnp.int32)
    table_t = _retile_table(jnp.swapaxes(table, 0, 1))
    return _sc_lookup(table_t, idx)


# speedup vs baseline: 13.9223x; 1.0074x over previous
import functools

import jax
import jax.numpy as jnp
from jax import lax
from jax.experimental import pallas as pl
from jax.experimental.pallas import tpu as pltpu
from jax.experimental.pallas import tpu_sc as plsc

_VOCAB = 100000
_VOCAB_PAD = 102400
_EMBED = 32
_B = 4
_D = 64
_HW = 64 * 64
_SPATIAL = _D * _HW
_NCHUNKS = _B * _D


def _retile_table(table_t2d):
    cols = 4096
    grid = _VOCAB_PAD // cols

    def body(in_ref, out_ref):
        out_ref[...] = in_ref[...].reshape(_EMBED, cols // 128, 128)

    return pl.pallas_call(
        body,
        grid=(grid,),
        in_specs=[pl.BlockSpec((_EMBED, cols), lambda g: (0, g))],
        out_specs=pl.BlockSpec(
            (_EMBED, cols // 128, 128), lambda g: (0, g, 0)
        ),
        out_shape=jax.ShapeDtypeStruct(
            (_EMBED, _VOCAB_PAD // 128, 128), jnp.float32
        ),
    )(table_t2d)


def _sc_lookup(table_t, idx):
    mesh = plsc.VectorSubcoreMesh(core_axis_name="c", subcore_axis_name="s")

    @functools.partial(
        pl.kernel,
        mesh=mesh,
        compiler_params=pltpu.CompilerParams(
            use_tc_tiling_on_sc=True, needs_layout_passes=False
        ),
        out_type=jax.ShapeDtypeStruct((_B, _EMBED, _D, 64, 64), jnp.float32),
        scratch_types=[
            pltpu.VMEM((_VOCAB_PAD // 128, 128), jnp.float32),
            pltpu.VMEM((32, 128), jnp.int32),
            pltpu.VMEM((32, 128), jnp.int32),
            pltpu.VMEM((64, 64), jnp.float32),
            pltpu.VMEM((64, 64), jnp.float32),
            pltpu.VMEM_SHARED((2, 32, 128), jnp.int32),
            pltpu.SemaphoreType.DMA,
            (pltpu.SemaphoreType.DMA,) * 2,
            (pltpu.SemaphoreType.DMA,) * 2,
            (pltpu.SemaphoreType.DMA,) * 2,
        ],
    )
    def k(table_t_hbm, idx_hbm, out_hbm, row_v, idx_v0, idx_v1,
          val_v0, val_v1, idx_sh, row_sem, lead_sems, in_sems, out_sems):
        c = lax.axis_index("c")
        s = lax.axis_index("s")
        e = s * 2 + c
        is_leader = s == 0
        row_copy = pltpu.make_async_copy(table_t_hbm.at[e], row_v, row_sem)
        row_copy.start()
        idx_bufs = (idx_v0, idx_v1)
        val_bufs = (val_v0, val_v1)

        def lead_fetch(g, p):
            pltpu.make_async_copy(
                idx_hbm.at[g], idx_sh.at[p], lead_sems[p]
            ).start()

        def lead_wait(g, p):
            pltpu.make_async_copy(
                idx_hbm.at[g], idx_sh.at[p], lead_sems[p]
            ).wait()

        def local_fetch(p):
            pltpu.make_async_copy(
                idx_sh.at[p], idx_bufs[p], in_sems[p]
            ).start()

        def local_wait(p):
            pltpu.make_async_copy(
                idx_sh.at[p], idx_bufs[p], in_sems[p]
            ).wait()

        def out_slot(g):
            b = g // _D
            d = g % _D
            return out_hbm.at[b, e, d]

        def start_out(g, buf, sem):
            pltpu.make_async_copy(buf, out_slot(g), sem).start()

        def wait_out(g, buf, sem):
            pltpu.make_async_copy(buf, out_slot(g), sem).wait()

        def gather(ibuf, vbuf):
            @plsc.parallel_loop(0, _HW // 16, unroll=16)
            def _(i):
                ids = ibuf[i // 8, pl.ds((i % 8) * 16, 16)]
                h = i // 4
                w0 = (i % 4) * 16
                vbuf[h, pl.ds(w0, 16)] = plsc.load_gather(
                    row_v, [ids >> 7, ids & 127]
                )

        @pl.when(is_leader)
        def _():
            lead_fetch(0, 0)
        row_copy.wait()

        @pl.when(is_leader)
        def _():
            lead_wait(0, 0)
        plsc.subcore_barrier()
        local_fetch(0)

        @pl.when(is_leader)
        def _():
            lead_fetch(1, 1)

        def chunk_pair(h, carry):
            g0 = 2 * h
            g1 = g0 + 1
            last = h + 1 == _NCHUNKS // 2

            @pl.when(is_leader)
            def _():
                lead_wait(g1, 1)
            local_wait(0)
            plsc.subcore_barrier()
            local_fetch(1)

            @pl.when(is_leader & jnp.logical_not(last))
            def _():
                lead_fetch(g0 + 2, 0)

            @pl.when(h > 0)
            def _():
                wait_out(g0 - 2, val_bufs[0], out_sems[0])
            gather(idx_bufs[0], val_bufs[0])
            start_out(g0, val_bufs[0], out_sems[0])

            @pl.when(is_leader & jnp.logical_not(last))
            def _():
                lead_wait(g0 + 2, 0)
            local_wait(1)
            plsc.subcore_barrier()

            @pl.when(jnp.logical_not(last))
            def _():
                local_fetch(0)

            @pl.when(is_leader & jnp.logical_not(last))
            def _():
                lead_fetch(g1 + 2, 1)

            @pl.when(h > 0)
            def _():
                wait_out(g1 - 2, val_bufs[1], out_sems[1])
            gather(idx_bufs[1], val_bufs[1])
            start_out(g1, val_bufs[1], out_sems[1])
            return carry

        lax.fori_loop(0, _NCHUNKS // 2, chunk_pair, 0)
        wait_out(_NCHUNKS - 2, val_bufs[0], out_sems[0])
        wait_out(_NCHUNKS - 1, val_bufs[1], out_sems[1])

    return k(table_t, idx)


def kernel(x, table):
    idx = x[:, 0].reshape(_NCHUNKS, 32, 128).astype(jnp.int32)
    table_t = _retile_table(jnp.swapaxes(table, 0, 1))
    return _sc_lookup(table_t, idx)
